# baseline (speedup 1.0000x reference)
import functools

import jax
import jax.numpy as jnp
from jax.experimental import pallas as pl
from jax.experimental.pallas import tpu as pltpu

_F32 = jnp.float32
_BF16 = jnp.bfloat16
_PAD = 8


def _lin1_body(x_ref, w_ref, b_ref, hTe_ref, hnT_ref, *, hid):
    h = jax.lax.dot_general(w_ref[...], x_ref[...], (((1,), (1,)), ((), ())),
                            preferred_element_type=_F32)
    h = jnp.maximum(h + b_ref[...], 0.0)
    hTe_ref[0:hid, :] = h.astype(_BF16)
    hTe_ref[hid:, :] = jnp.ones_like(hTe_ref[hid:, :])
    nrm = jnp.sqrt(jnp.sum(h * h, axis=0, keepdims=True))
    hnT_ref[...] = (h / jnp.maximum(nrm, 1e-12)).astype(_BF16)


def _diag_update(acc_ref, hTe, e, bs, bi):
    r = jax.lax.broadcasted_iota(jnp.int32, (bs, bi), 0)
    c = jax.lax.broadcasted_iota(jnp.int32, (bs, bi), 1)
    pd = jnp.where(r == c, e, _BF16(0.0))
    acc_ref[...] += jax.lax.dot_general(
        hTe, pd, (((1,), (0,)), ((), ())), preferred_element_type=_F32)


def _prop1_body(beta2_ref, adj_ref, hTe_ref, hnT_ref, hniT_ref,
                oTe_ref, onT_ref, obnT_ref, adj8_ref, acc_ref,
                *, bs, bi, hid):
    i = pl.program_id(0)
    s = pl.program_id(1)
    ns = pl.num_programs(1)

    @pl.when(s == 0)
    def _():
        acc_ref[...] = jnp.zeros_like(acc_ref)

    e = jnp.exp(jax.lax.dot_general(
        hnT_ref[...], hniT_ref[...], (((0,), (0,)), ((), ())),
        preferred_element_type=_F32).astype(_BF16))
    ab = adj_ref[...].astype(_BF16)
    adj8_ref[...] = ab
    p = ab * e
    acc_ref[...] += jax.lax.dot_general(
        hTe_ref[...], p, (((1,), (0,)), ((), ())),
        preferred_element_type=_F32)

    @pl.when(i == s)
    def _():
        _diag_update(acc_ref, hTe_ref[...], e, bs, bi)

    @pl.when(s == ns - 1)
    def _():
        o = acc_ref[0:hid, :] / acc_ref[hid:hid + 1, :]
        oTe_ref[0:hid, :] = o.astype(_BF16)
        oTe_ref[hid:, :] = jnp.ones_like(oTe_ref[hid:, :])
        nrm = jnp.sqrt(jnp.sum(o * o, axis=0, keepdims=True))
        on = o / jnp.maximum(nrm, 1e-12)
        onT_ref[...] = on.astype(_BF16)
        obnT_ref[...] = (beta2_ref[0] * on).astype(_BF16)


def _prop2_body(adj8_ref, hTe_ref, hnT_ref, hniT_ref, w2_ref, b2_ref,
                h4T_ref, acc_ref, *, bs, bi, hid):
    i = pl.program_id(0)
    s = pl.program_id(1)
    ns = pl.num_programs(1)

    @pl.when(s == 0)
    def _():
        acc_ref[...] = jnp.zeros_like(acc_ref)

    e = jnp.exp(jax.lax.dot_general(
        hnT_ref[...], hniT_ref[...], (((0,), (0,)), ((), ())),
        preferred_element_type=_F32).astype(_BF16))
    p = adj8_ref[...] * e
    acc_ref[...] += jax.lax.dot_general(
        hTe_ref[...], p, (((1,), (0,)), ((), ())),
        preferred_element_type=_F32)

    @pl.when(i == s)
    def _():
        _diag_update(acc_ref, hTe_ref[...], e, bs, bi)

    @pl.when(s == ns - 1)
    def _():
        o = acc_ref[0:hid, :] / acc_ref[hid:hid + 1, :]
        h4 = jax.lax.dot_general(
            w2_ref[...], o, (((1,), (0,)), ((), ())),
            preferred_element_type=_F32)
        h4T_ref[...] = jnp.maximum(h4 + b2_ref[...], 0.0).astype(_BF16)


def _adjmm_body(adj_ref, h4T_ref, out_ref):
    j = pl.program_id(1)

    @pl.when(j == 0)
    def _():
        out_ref[...] = jnp.zeros_like(out_ref)

    out_ref[...] += jax.lax.dot_general(
        adj_ref[...], h4T_ref[...], (((1,), (1,)), ((), ())),
        preferred_element_type=_F32)


def _impl(x, adj, W1, b1, W2, b2, beta2, interpret=False):
    n, in_ch = x.shape
    hid = W1.shape[0]
    he = hid + _PAD
    bn = min(512, n)
    bs = bi = min(1024, n)
    ni, ns = n // bi, n // bs

    b1c = b1.reshape(hid, 1)
    b2c = b2.reshape(hid, 1)

    hTe, hnT = pl.pallas_call(
        functools.partial(_lin1_body, hid=hid),
        grid=(n // bn,),
        in_specs=[
            pl.BlockSpec((bn, in_ch), lambda j: (j, 0)),
            pl.BlockSpec((hid, in_ch), lambda j: (0, 0)),
            pl.BlockSpec((hid, 1), lambda j: (0, 0)),
        ],
        out_specs=[pl.BlockSpec((he, bn), lambda j: (0, j)),
                   pl.BlockSpec((hid, bn), lambda j: (0, j))],
        out_shape=[jax.ShapeDtypeStruct((he, n), _BF16),
                   jax.ShapeDtypeStruct((hid, n), _BF16)],
        interpret=interpret,
    )(x, W1, b1c)

    h2Te, h2nT, h2bnT, adj_i8 = pl.pallas_call(
        functools.partial(_prop1_body, bs=bs, bi=bi, hid=hid),
        grid=(ni, ns),
        in_specs=[
            pl.BlockSpec(memory_space=pltpu.SMEM),
            pl.BlockSpec((bs, bi), lambda i, s: (s, i)),
            pl.BlockSpec((he, bs), lambda i, s: (0, s)),
            pl.BlockSpec((hid, bs), lambda i, s: (0, s)),
            pl.BlockSpec((hid, bi), lambda i, s: (0, i)),
        ],
        out_specs=[
            pl.BlockSpec((he, bi), lambda i, s: (0, i)),
            pl.BlockSpec((hid, bi), lambda i, s: (0, i)),
            pl.BlockSpec((hid, bi), lambda i, s: (0, i)),
            pl.BlockSpec((bs, bi), lambda i, s: (s, i)),
        ],
        out_shape=[
            jax.ShapeDtypeStruct((he, n), _BF16),
            jax.ShapeDtypeStruct((hid, n), _BF16),
            jax.ShapeDtypeStruct((hid, n), _BF16),
            jax.ShapeDtypeStruct((n, n), _BF16),
        ],
        scratch_shapes=[pltpu.VMEM((he, bi), _F32)],
        interpret=interpret,
    )(beta2.reshape(1).astype(_F32), adj, hTe, hnT, hnT)

    h4T = pl.pallas_call(
        functools.partial(_prop2_body, bs=bs, bi=bi, hid=hid),
        grid=(ni, ns),
        in_specs=[
            pl.BlockSpec((bs, bi), lambda i, s: (s, i)),
            pl.BlockSpec((he, bs), lambda i, s: (0, s)),
            pl.BlockSpec((hid, bs), lambda i, s: (0, s)),
            pl.BlockSpec((hid, bi), lambda i, s: (0, i)),
            pl.BlockSpec((hid, hid), lambda i, s: (0, 0)),
            pl.BlockSpec((hid, 1), lambda i, s: (0, 0)),
        ],
        out_specs=pl.BlockSpec((hid, bi), lambda i, s: (0, i)),
        out_shape=jax.ShapeDtypeStruct((hid, n), _BF16),
        scratch_shapes=[pltpu.VMEM((he, bi), _F32)],
        interpret=interpret,
    )(adj_i8, h2Te, h2nT, h2bnT, W2, b2c)

    bi4 = bj4 = min(1024, n)
    out = pl.pallas_call(
        _adjmm_body,
        grid=(n // bi4, n // bj4),
        in_specs=[
            pl.BlockSpec((bi4, bj4), lambda i, j: (i, j)),
            pl.BlockSpec((hid, bj4), lambda i, j: (0, j)),
        ],
        out_specs=pl.BlockSpec((bi4, hid), lambda i, j: (i, 0)),
        out_shape=jax.ShapeDtypeStruct((n, hid), _F32),
        interpret=interpret,
    )(adj_i8, h4T)
    return out


def kernel(x, adj, W1, b1, W2, b2, beta2):
    return _impl(x, adj, W1, b1, W2, b2, beta2)


# speedup vs baseline: 1.1215x; 1.1215x over previous
import functools

import jax
import jax.numpy as jnp
from jax.experimental import pallas as pl
from jax.experimental.pallas import tpu as pltpu

_F32 = jnp.float32
_BF16 = jnp.bfloat16
_PAD = 8


def _lin1_body(x_ref, w_ref, b_ref, hTe_ref, hnT_ref, *, hid):
    h = jax.lax.dot_general(w_ref[...], x_ref[...], (((1,), (1,)), ((), ())),
                            preferred_element_type=_F32)
    h = jnp.maximum(h + b_ref[...], 0.0)
    hTe_ref[0:hid, :] = h.astype(_BF16)
    hTe_ref[hid:, :] = jnp.ones_like(hTe_ref[hid:, :])
    nrm = jnp.sqrt(jnp.sum(h * h, axis=0, keepdims=True))
    hnT_ref[...] = (h / jnp.maximum(nrm, 1e-12)).astype(_BF16)


def _diag_update(acc_ref, hTe, e, bs, bi):
    r = jax.lax.broadcasted_iota(jnp.int32, (bs, bi), 0)
    c = jax.lax.broadcasted_iota(jnp.int32, (bs, bi), 1)
    pd = jnp.where(r == c, e, _BF16(0.0))
    acc_ref[...] += jax.lax.dot_general(
        hTe, pd, (((1,), (0,)), ((), ())), preferred_element_type=_F32)


def _prop1_body(beta2_ref, adj_ref, hTe_ref, hnT_ref, hniT_ref,
                oTe_ref, onT_ref, obnT_ref, adj8_ref, acc_ref,
                *, bs, bi, hid):
    i = pl.program_id(0)
    s = pl.program_id(1)
    ns = pl.num_programs(1)

    @pl.when(s == 0)
    def _():
        acc_ref[...] = jnp.zeros_like(acc_ref)

    e = jnp.exp(jax.lax.dot_general(
        hnT_ref[...], hniT_ref[...], (((0,), (0,)), ((), ())),
        preferred_element_type=_F32).astype(_BF16))
    a = adj_ref[...]
    adj8_ref[...] = a.astype(jnp.int8)
    p = a.astype(_BF16) * e
    acc_ref[...] += jax.lax.dot_general(
        hTe_ref[...], p, (((1,), (0,)), ((), ())),
        preferred_element_type=_F32)

    @pl.when(i == s)
    def _():
        _diag_update(acc_ref, hTe_ref[...], e, bs, bi)

    @pl.when(s == ns - 1)
    def _():
        o = acc_ref[0:hid, :] / acc_ref[hid:hid + 1, :]
        oTe_ref[0:hid, :] = o.astype(_BF16)
        oTe_ref[hid:, :] = jnp.ones_like(oTe_ref[hid:, :])
        nrm = jnp.sqrt(jnp.sum(o * o, axis=0, keepdims=True))
        on = o / jnp.maximum(nrm, 1e-12)
        onT_ref[...] = on.astype(_BF16)
        obnT_ref[...] = (beta2_ref[0] * on).astype(_BF16)


def _prop2_body(adj8_ref, hTe_ref, hnT_ref, hniT_ref, w2_ref, b2_ref,
                h4T_ref, acc_ref, *, bs, bi, hid):
    i = pl.program_id(0)
    s = pl.program_id(1)
    ns = pl.num_programs(1)

    @pl.when(s == 0)
    def _():
        acc_ref[...] = jnp.zeros_like(acc_ref)

    e = jnp.exp(jax.lax.dot_general(
        hnT_ref[...], hniT_ref[...], (((0,), (0,)), ((), ())),
        preferred_element_type=_F32).astype(_BF16))
    p = jnp.where(adj8_ref[...] != 0, e, _BF16(0.0))
    acc_ref[...] += jax.lax.dot_general(
        hTe_ref[...], p, (((1,), (0,)), ((), ())),
        preferred_element_type=_F32)

    @pl.when(i == s)
    def _():
        _diag_update(acc_ref, hTe_ref[...], e, bs, bi)

    @pl.when(s == ns - 1)
    def _():
        o = acc_ref[0:hid, :] / acc_ref[hid:hid + 1, :]
        h4 = jax.lax.dot_general(
            w2_ref[...], o, (((1,), (0,)), ((), ())),
            preferred_element_type=_F32)
        h4T_ref[...] = jnp.maximum(h4 + b2_ref[...], 0.0).astype(_BF16)


def _adjmm_body(adj_ref, h4T_ref, out_ref):
    j = pl.program_id(1)

    @pl.when(j == 0)
    def _():
        out_ref[...] = jnp.zeros_like(out_ref)

    out_ref[...] += jax.lax.dot_general(
        adj_ref[...].astype(_BF16), h4T_ref[...], (((1,), (1,)), ((), ())),
        preferred_element_type=_F32)


def _impl(x, adj, W1, b1, W2, b2, beta2, interpret=False):
    n, in_ch = x.shape
    hid = W1.shape[0]
    he = hid + _PAD
    bn = min(512, n)
    bs = bi = min(1024, n)
    ni, ns = n // bi, n // bs

    b1c = b1.reshape(hid, 1)
    b2c = b2.reshape(hid, 1)

    hTe, hnT = pl.pallas_call(
        functools.partial(_lin1_body, hid=hid),
        grid=(n // bn,),
        in_specs=[
            pl.BlockSpec((bn, in_ch), lambda j: (j, 0)),
            pl.BlockSpec((hid, in_ch), lambda j: (0, 0)),
            pl.BlockSpec((hid, 1), lambda j: (0, 0)),
        ],
        out_specs=[pl.BlockSpec((he, bn), lambda j: (0, j)),
                   pl.BlockSpec((hid, bn), lambda j: (0, j))],
        out_shape=[jax.ShapeDtypeStruct((he, n), _BF16),
                   jax.ShapeDtypeStruct((hid, n), _BF16)],
        interpret=interpret,
    )(x, W1, b1c)

    h2Te, h2nT, h2bnT, adj_i8 = pl.pallas_call(
        functools.partial(_prop1_body, bs=bs, bi=bi, hid=hid),
        grid=(ni, ns),
        in_specs=[
            pl.BlockSpec(memory_space=pltpu.SMEM),
            pl.BlockSpec((bs, bi), lambda i, s: (s, i)),
            pl.BlockSpec((he, bs), lambda i, s: (0, s)),
            pl.BlockSpec((hid, bs), lambda i, s: (0, s)),
            pl.BlockSpec((hid, bi), lambda i, s: (0, i)),
        ],
        out_specs=[
            pl.BlockSpec((he, bi), lambda i, s: (0, i)),
            pl.BlockSpec((hid, bi), lambda i, s: (0, i)),
            pl.BlockSpec((hid, bi), lambda i, s: (0, i)),
            pl.BlockSpec((bs, bi), lambda i, s: (s, i)),
        ],
        out_shape=[
            jax.ShapeDtypeStruct((he, n), _BF16),
            jax.ShapeDtypeStruct((hid, n), _BF16),
            jax.ShapeDtypeStruct((hid, n), _BF16),
            jax.ShapeDtypeStruct((n, n), jnp.int8),
        ],
        scratch_shapes=[pltpu.VMEM((he, bi), _F32)],
        interpret=interpret,
    )(beta2.reshape(1).astype(_F32), adj, hTe, hnT, hnT)

    h4T = pl.pallas_call(
        functools.partial(_prop2_body, bs=bs, bi=bi, hid=hid),
        grid=(ni, ns),
        in_specs=[
            pl.BlockSpec((bs, bi), lambda i, s: (s, i)),
            pl.BlockSpec((he, bs), lambda i, s: (0, s)),
            pl.BlockSpec((hid, bs), lambda i, s: (0, s)),
            pl.BlockSpec((hid, bi), lambda i, s: (0, i)),
            pl.BlockSpec((hid, hid), lambda i, s: (0, 0)),
            pl.BlockSpec((hid, 1), lambda i, s: (0, 0)),
        ],
        out_specs=pl.BlockSpec((hid, bi), lambda i, s: (0, i)),
        out_shape=jax.ShapeDtypeStruct((hid, n), _BF16),
        scratch_shapes=[pltpu.VMEM((he, bi), _F32)],
        interpret=interpret,
    )(adj_i8, h2Te, h2nT, h2bnT, W2, b2c)

    bi4 = bj4 = min(1024, n)
    out = pl.pallas_call(
        _adjmm_body,
        grid=(n // bi4, n // bj4),
        in_specs=[
            pl.BlockSpec((bi4, bj4), lambda i, j: (i, j)),
            pl.BlockSpec((hid, bj4), lambda i, j: (0, j)),
        ],
        out_specs=pl.BlockSpec((bi4, hid), lambda i, j: (i, 0)),
        out_shape=jax.ShapeDtypeStruct((n, hid), _F32),
        interpret=interpret,
    )(adj_i8, h4T)
    return out


def kernel(x, adj, W1, b1, W2, b2, beta2):
    return _impl(x, adj, W1, b1, W2, b2, beta2)


# speedup vs baseline: 1.2577x; 1.1215x over previous
import functools

import jax
import jax.numpy as jnp
from jax.experimental import pallas as pl
from jax.experimental.pallas import tpu as pltpu

_F32 = jnp.float32
_BF16 = jnp.bfloat16
_PAD = 8


def _lin1_body(x_ref, w_ref, b_ref, hTe_ref, hnT_ref, *, hid):
    h = jax.lax.dot_general(w_ref[...], x_ref[...], (((1,), (1,)), ((), ())),
                            preferred_element_type=_F32)
    h = jnp.maximum(h + b_ref[...], 0.0)
    hTe_ref[0:hid, :] = h.astype(_BF16)
    hTe_ref[hid:, :] = jnp.ones_like(hTe_ref[hid:, :])
    nrm = jnp.sqrt(jnp.sum(h * h, axis=0, keepdims=True))
    hnT_ref[...] = (h / jnp.maximum(nrm, 1e-12)).astype(_BF16)


def _diag_update(acc_ref, hTe, e, bs, bi):
    r = jax.lax.broadcasted_iota(jnp.int32, (bs, bi), 0)
    c = jax.lax.broadcasted_iota(jnp.int32, (bs, bi), 1)
    pd = jnp.where(r == c, e, _BF16(0.0))
    acc_ref[...] += jax.lax.dot_general(
        hTe, pd, (((1,), (0,)), ((), ())), preferred_element_type=_F32)


def _prop1_body(beta2_ref, adj_ref, hTe_ref, hnT_ref, hniT_ref,
                oTe_ref, onT_ref, obnT_ref, adj8_ref, acc_ref,
                *, bs, bi, hid):
    i = pl.program_id(0)
    s = pl.program_id(1)
    ns = pl.num_programs(1)

    @pl.when(s == 0)
    def _():
        acc_ref[...] = jnp.zeros_like(acc_ref)

    e = jnp.exp(jax.lax.dot_general(
        hnT_ref[...], hniT_ref[...], (((0,), (0,)), ((), ())),
        preferred_element_type=_F32).astype(_BF16))
    a = adj_ref[...]
    adj8_ref[...] = a.astype(jnp.int8)
    p = a.astype(_BF16) * e
    acc_ref[...] += jax.lax.dot_general(
        hTe_ref[...], p, (((1,), (0,)), ((), ())),
        preferred_element_type=_F32)

    @pl.when(i == s)
    def _():
        _diag_update(acc_ref, hTe_ref[...], e, bs, bi)

    @pl.when(s == ns - 1)
    def _():
        o = acc_ref[0:hid, :] / acc_ref[hid:hid + 1, :]
        oTe_ref[0:hid, :] = o.astype(_BF16)
        oTe_ref[hid:, :] = jnp.ones_like(oTe_ref[hid:, :])
        nrm = jnp.sqrt(jnp.sum(o * o, axis=0, keepdims=True))
        on = o / jnp.maximum(nrm, 1e-12)
        onT_ref[...] = on.astype(_BF16)
        obnT_ref[...] = (beta2_ref[0] * on).astype(_BF16)


def _prop2_body(adj8_ref, hTe_ref, hnT_ref, hniT_ref, w2_ref, b2_ref,
                h4T_ref, acc_ref, *, bs, bi, hid):
    i = pl.program_id(0)
    s = pl.program_id(1)
    ns = pl.num_programs(1)

    @pl.when(s == 0)
    def _():
        acc_ref[...] = jnp.zeros_like(acc_ref)

    e = jnp.exp(jax.lax.dot_general(
        hnT_ref[...], hniT_ref[...], (((0,), (0,)), ((), ())),
        preferred_element_type=_F32).astype(_BF16))
    p = jnp.where(adj8_ref[...] != 0, e, _BF16(0.0))
    acc_ref[...] += jax.lax.dot_general(
        hTe_ref[...], p, (((1,), (0,)), ((), ())),
        preferred_element_type=_F32)

    @pl.when(i == s)
    def _():
        _diag_update(acc_ref, hTe_ref[...], e, bs, bi)

    @pl.when(s == ns - 1)
    def _():
        o = acc_ref[0:hid, :] / acc_ref[hid:hid + 1, :]
        h4 = jax.lax.dot_general(
            w2_ref[...], o, (((1,), (0,)), ((), ())),
            preferred_element_type=_F32)
        h4T_ref[...] = jnp.maximum(h4 + b2_ref[...], 0.0).astype(_BF16)


def _adjmm_body(adj_ref, h4T_ref, out_ref):
    j = pl.program_id(1)

    @pl.when(j == 0)
    def _():
        out_ref[...] = jnp.zeros_like(out_ref)

    out_ref[...] += jax.lax.dot_general(
        adj_ref[...].astype(_BF16), h4T_ref[...], (((1,), (1,)), ((), ())),
        preferred_element_type=_F32)


def _impl(x, adj, W1, b1, W2, b2, beta2, interpret=False):
    n, in_ch = x.shape
    hid = W1.shape[0]
    he = hid + _PAD
    bn = min(512, n)
    bs = bi = min(1024, n)
    bs2 = bi2 = min(2048, n)
    ni, ns = n // bi, n // bs
    ni2, ns2 = n // bi2, n // bs2

    b1c = b1.reshape(hid, 1)
    b2c = b2.reshape(hid, 1)

    hTe, hnT = pl.pallas_call(
        functools.partial(_lin1_body, hid=hid),
        grid=(n // bn,),
        in_specs=[
            pl.BlockSpec((bn, in_ch), lambda j: (j, 0)),
            pl.BlockSpec((hid, in_ch), lambda j: (0, 0)),
            pl.BlockSpec((hid, 1), lambda j: (0, 0)),
        ],
        out_specs=[pl.BlockSpec((he, bn), lambda j: (0, j)),
                   pl.BlockSpec((hid, bn), lambda j: (0, j))],
        out_shape=[jax.ShapeDtypeStruct((he, n), _BF16),
                   jax.ShapeDtypeStruct((hid, n), _BF16)],
        interpret=interpret,
    )(x, W1, b1c)

    h2Te, h2nT, h2bnT, adj_i8 = pl.pallas_call(
        functools.partial(_prop1_body, bs=bs, bi=bi, hid=hid),
        grid=(ni, ns),
        in_specs=[
            pl.BlockSpec(memory_space=pltpu.SMEM),
            pl.BlockSpec((bs, bi), lambda i, s: (s, i)),
            pl.BlockSpec((he, bs), lambda i, s: (0, s)),
            pl.BlockSpec((hid, bs), lambda i, s: (0, s)),
            pl.BlockSpec((hid, bi), lambda i, s: (0, i)),
        ],
        out_specs=[
            pl.BlockSpec((he, bi), lambda i, s: (0, i)),
            pl.BlockSpec((hid, bi), lambda i, s: (0, i)),
            pl.BlockSpec((hid, bi), lambda i, s: (0, i)),
            pl.BlockSpec((bs, bi), lambda i, s: (s, i)),
        ],
        out_shape=[
            jax.ShapeDtypeStruct((he, n), _BF16),
            jax.ShapeDtypeStruct((hid, n), _BF16),
            jax.ShapeDtypeStruct((hid, n), _BF16),
            jax.ShapeDtypeStruct((n, n), jnp.int8),
        ],
        scratch_shapes=[pltpu.VMEM((he, bi), _F32)],
        interpret=interpret,
    )(beta2.reshape(1).astype(_F32), adj, hTe, hnT, hnT)

    h4T = pl.pallas_call(
        functools.partial(_prop2_body, bs=bs2, bi=bi2, hid=hid),
        grid=(ni2, ns2),
        in_specs=[
            pl.BlockSpec((bs2, bi2), lambda i, s: (s, i)),
            pl.BlockSpec((he, bs2), lambda i, s: (0, s)),
            pl.BlockSpec((hid, bs2), lambda i, s: (0, s)),
            pl.BlockSpec((hid, bi2), lambda i, s: (0, i)),
            pl.BlockSpec((hid, hid), lambda i, s: (0, 0)),
            pl.BlockSpec((hid, 1), lambda i, s: (0, 0)),
        ],
        out_specs=pl.BlockSpec((hid, bi2), lambda i, s: (0, i)),
        out_shape=jax.ShapeDtypeStruct((hid, n), _BF16),
        scratch_shapes=[pltpu.VMEM((he, bi2), _F32)],
        interpret=interpret,
    )(adj_i8, h2Te, h2nT, h2bnT, W2, b2c)

    bi4 = min(1024, n)
    bj4 = min(2048, n)
    out = pl.pallas_call(
        _adjmm_body,
        grid=(n // bi4, n // bj4),
        in_specs=[
            pl.BlockSpec((bi4, bj4), lambda i, j: (i, j)),
            pl.BlockSpec((hid, bj4), lambda i, j: (0, j)),
        ],
        out_specs=pl.BlockSpec((bi4, hid), lambda i, j: (i, 0)),
        out_shape=jax.ShapeDtypeStruct((n, hid), _F32),
        interpret=interpret,
    )(adj_i8, h4T)
    return out


def kernel(x, adj, W1, b1, W2, b2, beta2):
    return _impl(x, adj, W1, b1, W2, b2, beta2)


# speedup vs baseline: 1.2984x; 1.0323x over previous
import functools

import jax
import jax.numpy as jnp
from jax.experimental import pallas as pl
from jax.experimental.pallas import tpu as pltpu

_F32 = jnp.float32
_BF16 = jnp.bfloat16
_PAD = 8


def _lin1_body(x_ref, w_ref, b_ref, hTe_ref, hnT_ref, *, hid):
    h = jax.lax.dot_general(w_ref[...], x_ref[...], (((1,), (1,)), ((), ())),
                            preferred_element_type=_F32)
    h = jnp.maximum(h + b_ref[...], 0.0)
    hTe_ref[0:hid, :] = h.astype(_BF16)
    hTe_ref[hid:, :] = jnp.ones_like(hTe_ref[hid:, :])
    nrm = jnp.sqrt(jnp.sum(h * h, axis=0, keepdims=True))
    hnT_ref[...] = (h / jnp.maximum(nrm, 1e-12)).astype(_BF16)


def _diag_update(acc_ref, hTe, e, bs, bi):
    r = jax.lax.broadcasted_iota(jnp.int32, (bs, bi), 0)
    c = jax.lax.broadcasted_iota(jnp.int32, (bs, bi), 1)
    pd = jnp.where(r == c, e, _BF16(0.0))
    acc_ref[...] += jax.lax.dot_general(
        hTe, pd, (((1,), (0,)), ((), ())), preferred_element_type=_F32)


def _prop1_body(beta2_ref, adj_ref, hTe_ref, hnT_ref, hniT_ref,
                oTe_ref, onT_ref, obnT_ref, adj8_ref, acc_ref,
                *, bs, bi, hid):
    i = pl.program_id(0)
    s = pl.program_id(1)
    ns = pl.num_programs(1)

    @pl.when(s == 0)
    def _():
        acc_ref[...] = jnp.zeros_like(acc_ref)

    e = jnp.exp(jax.lax.dot_general(
        hnT_ref[...], hniT_ref[...], (((0,), (0,)), ((), ())),
        preferred_element_type=_F32).astype(_BF16))
    a = adj_ref[...]
    adj8_ref[...] = a.astype(jnp.int8)
    p = a.astype(_BF16) * e
    acc_ref[...] += jax.lax.dot_general(
        hTe_ref[...], p, (((1,), (0,)), ((), ())),
        preferred_element_type=_F32)

    @pl.when(i == s)
    def _():
        _diag_update(acc_ref, hTe_ref[...], e, bs, bi)

    @pl.when(s == ns - 1)
    def _():
        o = acc_ref[0:hid, :] / acc_ref[hid:hid + 1, :]
        oTe_ref[0:hid, :] = o.astype(_BF16)
        oTe_ref[hid:, :] = jnp.ones_like(oTe_ref[hid:, :])
        nrm = jnp.sqrt(jnp.sum(o * o, axis=0, keepdims=True))
        on = o / jnp.maximum(nrm, 1e-12)
        onT_ref[...] = on.astype(_BF16)
        obnT_ref[...] = (beta2_ref[0] * on).astype(_BF16)


def _prop2_body(adj8_ref, hTe_ref, hnT_ref, hniT_ref, w2_ref, b2_ref,
                h4T_ref, acc_ref, *, bs, bi, hid):
    i = pl.program_id(0)
    s = pl.program_id(1)
    ns = pl.num_programs(1)

    @pl.when(s == 0)
    def _():
        acc_ref[...] = jnp.zeros_like(acc_ref)

    e = jnp.exp(jax.lax.dot_general(
        hnT_ref[...], hniT_ref[...], (((0,), (0,)), ((), ())),
        preferred_element_type=_F32).astype(_BF16))
    p = jnp.where(adj8_ref[...] != 0, e, _BF16(0.0))
    acc_ref[...] += jax.lax.dot_general(
        hTe_ref[...], p, (((1,), (0,)), ((), ())),
        preferred_element_type=_F32)

    @pl.when(i == s)
    def _():
        _diag_update(acc_ref, hTe_ref[...], e, bs, bi)

    @pl.when(s == ns - 1)
    def _():
        o = acc_ref[0:hid, :] / acc_ref[hid:hid + 1, :]
        h4 = jax.lax.dot_general(
            w2_ref[...], o, (((1,), (0,)), ((), ())),
            preferred_element_type=_F32)
        h4T_ref[...] = jnp.maximum(h4 + b2_ref[...], 0.0).astype(_BF16)


def _adjmm_body(adj_ref, h4T_ref, out_ref):
    j = pl.program_id(1)

    @pl.when(j == 0)
    def _():
        out_ref[...] = jnp.zeros_like(out_ref)

    out_ref[...] += jax.lax.dot_general(
        adj_ref[...].astype(_BF16), h4T_ref[...], (((1,), (1,)), ((), ())),
        preferred_element_type=_F32)


def _impl(x, adj, W1, b1, W2, b2, beta2, interpret=False):
    n, in_ch = x.shape
    hid = W1.shape[0]
    he = hid + _PAD
    bn = min(512, n)
    bs = bi = min(1024, n)
    bs2 = bi2 = min(2048, n)
    ni, ns = n // bi, n // bs
    ni2, ns2 = n // bi2, n // bs2

    b1c = b1.reshape(hid, 1)
    b2c = b2.reshape(hid, 1)

    hTe, hnT = pl.pallas_call(
        functools.partial(_lin1_body, hid=hid),
        grid=(n // bn,),
        in_specs=[
            pl.BlockSpec((bn, in_ch), lambda j: (j, 0)),
            pl.BlockSpec((hid, in_ch), lambda j: (0, 0)),
            pl.BlockSpec((hid, 1), lambda j: (0, 0)),
        ],
        out_specs=[pl.BlockSpec((he, bn), lambda j: (0, j)),
                   pl.BlockSpec((hid, bn), lambda j: (0, j))],
        out_shape=[jax.ShapeDtypeStruct((he, n), _BF16),
                   jax.ShapeDtypeStruct((hid, n), _BF16)],
        interpret=interpret,
    )(x, W1, b1c)

    h2Te, h2nT, h2bnT, adj_i8 = pl.pallas_call(
        functools.partial(_prop1_body, bs=bs, bi=bi, hid=hid),
        grid=(ni, ns),
        in_specs=[
            pl.BlockSpec(memory_space=pltpu.SMEM),
            pl.BlockSpec((bs, bi), lambda i, s: (s, i)),
            pl.BlockSpec((he, bs), lambda i, s: (0, s)),
            pl.BlockSpec((hid, bs), lambda i, s: (0, s)),
            pl.BlockSpec((hid, bi), lambda i, s: (0, i)),
        ],
        out_specs=[
            pl.BlockSpec((he, bi), lambda i, s: (0, i)),
            pl.BlockSpec((hid, bi), lambda i, s: (0, i)),
            pl.BlockSpec((hid, bi), lambda i, s: (0, i)),
            pl.BlockSpec((bs, bi), lambda i, s: (s, i)),
        ],
        out_shape=[
            jax.ShapeDtypeStruct((he, n), _BF16),
            jax.ShapeDtypeStruct((hid, n), _BF16),
            jax.ShapeDtypeStruct((hid, n), _BF16),
            jax.ShapeDtypeStruct((n, n), jnp.int8),
        ],
        scratch_shapes=[pltpu.VMEM((he, bi), _F32)],
        interpret=interpret,
    )(beta2.reshape(1).astype(_F32), adj, hTe, hnT, hnT)

    h4T = pl.pallas_call(
        functools.partial(_prop2_body, bs=bs2, bi=bi2, hid=hid),
        grid=(ni2, ns2),
        in_specs=[
            pl.BlockSpec((bs2, bi2), lambda i, s: (s, i)),
            pl.BlockSpec((he, bs2), lambda i, s: (0, s)),
            pl.BlockSpec((hid, bs2), lambda i, s: (0, s)),
            pl.BlockSpec((hid, bi2), lambda i, s: (0, i)),
            pl.BlockSpec((hid, hid), lambda i, s: (0, 0)),
            pl.BlockSpec((hid, 1), lambda i, s: (0, 0)),
        ],
        out_specs=pl.BlockSpec((hid, bi2), lambda i, s: (0, i)),
        out_shape=jax.ShapeDtypeStruct((hid, n), _BF16),
        scratch_shapes=[pltpu.VMEM((he, bi2), _F32)],
        interpret=interpret,
    )(adj_i8, h2Te, h2nT, h2bnT, W2, b2c)

    bi4 = min(1024, n)
    bj4 = min(4096, n)
    out = pl.pallas_call(
        _adjmm_body,
        grid=(n // bi4, n // bj4),
        in_specs=[
            pl.BlockSpec((bi4, bj4), lambda i, j: (i, j)),
            pl.BlockSpec((hid, bj4), lambda i, j: (0, j)),
        ],
        out_specs=pl.BlockSpec((bi4, hid), lambda i, j: (i, 0)),
        out_shape=jax.ShapeDtypeStruct((n, hid), _F32),
        interpret=interpret,
    )(adj_i8, h4T)
    return out


def kernel(x, adj, W1, b1, W2, b2, beta2):
    return _impl(x, adj, W1, b1, W2, b2, beta2)
